# baseline (speedup 1.0000x reference)
import functools

import jax
import jax.numpy as jnp
import numpy as np
from jax import lax
from jax.experimental import pallas as pl
from jax.experimental.pallas import tpu as pltpu
from jax.experimental.pallas import tpu_sc as plsc

B = 1024
V = 100000
L = 50

NC = 2
NS = 16
NW = NC * NS
PER_W = B * L // NW
CHUNK = 128
LANES = 16

_CHUNKS = []
_off = 0
while _off < PER_W:
    _c = min(CHUNK, PER_W - _off)
    _CHUNKS.append((_off, _c))
    _off += _c

_ri = np.arange(B * L, dtype=np.int64) // L
_ROW_TERM = np.asarray(((_ri >> 7) << 10) + (_ri & 127), dtype=np.int32)

_mesh = plsc.VectorSubcoreMesh(core_axis_name="c", subcore_axis_name="s")


@functools.partial(
    pl.kernel,
    out_type=jax.ShapeDtypeStruct((NW, LANES), jnp.float32),
    mesh=_mesh,
    scratch_types=[
        pltpu.VMEM((PER_W,), jnp.int32),
        pltpu.VMEM((PER_W,), jnp.int32),
        pltpu.VMEM((PER_W,), jnp.float32),
        pltpu.VMEM((PER_W,), jnp.int32),
        pltpu.VMEM((PER_W,), jnp.float32),
        pltpu.VMEM((LANES,), jnp.float32),
        pltpu.SemaphoreType.DMA,
        pltpu.SemaphoreType.DMA,
    ],
)
def _pg_gather_mac(tgt_hbm, row_hbm, rew_hbm, pred_hbm, out_hbm,
                   tgt_v, row_v, rew_v, idx_v, val_v, acc_v, sem_in, sem_g):
    wid = lax.axis_index("s") * NC + lax.axis_index("c")
    base = wid * PER_W
    tgt_cp = pltpu.async_copy(tgt_hbm.at[pl.ds(base, PER_W)], tgt_v, sem_in)
    row_cp = pltpu.async_copy(row_hbm.at[pl.ds(base, PER_W)], row_v, sem_in)
    rew_cp = pltpu.async_copy(rew_hbm.at[pl.ds(base, PER_W)], rew_v, sem_in)
    tgt_cp.wait()
    row_cp.wait()
    copies = []
    for o, c in _CHUNKS:
        for r in range(o, o + c, LANES):
            t = tgt_v[pl.ds(r, LANES)]
            idx_v[pl.ds(r, LANES)] = (
                ((t >> 3) << 13) + ((t & 7) << 7) + row_v[pl.ds(r, LANES)]
            )
        copies.append(
            pltpu.async_copy(pred_hbm.at[idx_v.at[pl.ds(o, c)]],
                             val_v.at[pl.ds(o, c)], sem_g)
        )
    rew_cp.wait()
    acc = jnp.zeros((LANES,), jnp.float32)
    for (o, c), cp in zip(_CHUNKS, copies):
        cp.wait()
        for r in range(o, o + c, LANES):
            acc = acc + val_v[pl.ds(r, LANES)] * rew_v[pl.ds(r, LANES)]
    acc_v[...] = acc * jnp.float32(-1.0 / B)
    pltpu.sync_copy(acc_v, out_hbm.at[wid])


def kernel(pred, target, reward):
    tgt = target.astype(jnp.int32).reshape(-1)
    rew = reward.astype(jnp.float32).reshape(-1)
    row_term = jnp.asarray(_ROW_TERM)
    pred_lin = pred.reshape(8, 128, V // 8, 8).transpose(2, 0, 3, 1).reshape(-1)
    partial = _pg_gather_mac(tgt, row_term, rew, pred_lin)
    return jnp.sum(partial)


# speedup vs baseline: 1.0625x; 1.0625x over previous
import functools

import jax
import jax.numpy as jnp
from jax import lax
from jax.experimental import pallas as pl
from jax.experimental.pallas import tpu as pltpu
from jax.experimental.pallas import tpu_sc as plsc

B = 1024
V = 100000
L = 50

NC = 2
NS = 16
NW = NC * NS
PER_W = B * L // NW
CHUNK = 128
LANES = 16

_CHUNKS = []
_off = 0
while _off < PER_W:
    _c = min(CHUNK, PER_W - _off)
    _CHUNKS.append((_off, _c))
    _off += _c

_mesh = plsc.VectorSubcoreMesh(core_axis_name="c", subcore_axis_name="s")


@functools.partial(
    pl.kernel,
    out_type=jax.ShapeDtypeStruct((B * L,), jnp.float32),
    mesh=_mesh,
    scratch_types=[
        pltpu.VMEM((PER_W,), jnp.int32),
        pltpu.VMEM((PER_W,), jnp.float32),
        pltpu.SemaphoreType.DMA,
    ],
)
def _pg_gather(idx_hbm, pred_hbm, out_hbm, idx_v, val_v, sem_g):
    wid = lax.axis_index("s") * NC + lax.axis_index("c")
    base = wid * PER_W
    pltpu.sync_copy(idx_hbm.at[pl.ds(base, PER_W)], idx_v)
    copies = [
        pltpu.async_copy(pred_hbm.at[idx_v.at[pl.ds(o, c)]],
                         val_v.at[pl.ds(o, c)], sem_g)
        for o, c in _CHUNKS
    ]
    for cp in copies:
        cp.wait()
    pltpu.sync_copy(val_v, out_hbm.at[pl.ds(base, PER_W)])


def kernel(pred, target, reward):
    t = target.astype(jnp.int32)
    i = jnp.arange(B, dtype=jnp.int32)[:, None]
    n = ((t >> 3) << 13) + ((i >> 7) << 10) + ((t & 7) << 7) + (i & 127)
    pred_lin = pred.reshape(8, 128, V // 8, 8).transpose(2, 0, 3, 1).reshape(-1)
    val = _pg_gather(n.reshape(-1), pred_lin)
    return jnp.sum(val.reshape(B, L) * reward) * jnp.float32(-1.0 / B)


# speedup vs baseline: 1.0898x; 1.0257x over previous
import functools

import jax
import jax.numpy as jnp
from jax import lax
from jax.experimental import pallas as pl
from jax.experimental.pallas import tpu as pltpu
from jax.experimental.pallas import tpu_sc as plsc

B = 1024
V = 100000
L = 50

NC = 2
NS = 16
NW = NC * NS
PER_W = B * L // NW
CHUNK = 128
LANES = 16

_CHUNKS = []
_off = 0
while _off < PER_W:
    _c = min(CHUNK, PER_W - _off)
    _CHUNKS.append((_off, _c))
    _off += _c

_mesh = plsc.VectorSubcoreMesh(core_axis_name="c", subcore_axis_name="s")


@functools.partial(
    pl.kernel,
    out_type=jax.ShapeDtypeStruct((B * L,), jnp.float32),
    mesh=_mesh,
    scratch_types=[
        pltpu.VMEM((PER_W,), jnp.int32),
        pltpu.VMEM((PER_W,), jnp.float32),
        pltpu.SemaphoreType.DMA,
    ],
)
def _pg_gather(idx_hbm, pred_hbm, out_hbm, idx_v, val_v, sem_g):
    wid = lax.axis_index("s") * NC + lax.axis_index("c")
    base = wid * PER_W
    pltpu.sync_copy(idx_hbm.at[pl.ds(base, PER_W)], idx_v)
    copies = [
        pltpu.async_copy(pred_hbm.at[idx_v.at[pl.ds(o, c)]],
                         val_v.at[pl.ds(o, c)], sem_g)
        for o, c in _CHUNKS
    ]
    for cp in copies:
        cp.wait()
    pltpu.sync_copy(val_v, out_hbm.at[pl.ds(base, PER_W)])


def kernel(pred, target, reward):
    t = target.astype(jnp.int32)
    i = jnp.arange(B, dtype=jnp.int32)[:, None]
    n = ((t >> 3) << 13) + ((i >> 7) << 10) + ((t & 7) << 7) + (i & 127)
    pred_lin = pred.reshape(8, 128, V // 8, 8).transpose(2, 0, 3, 1).reshape(-1)
    val = _pg_gather(n.reshape(-1), pred_lin)
    return jnp.sum(val * reward.reshape(-1)) * jnp.float32(-1.0 / B)
